# baseline (speedup 1.0000x reference)
import functools

import numpy as np
import jax
import jax.numpy as jnp
from jax import lax
from jax.experimental import pallas as pl
from jax.experimental.pallas import tpu as pltpu
from jax.experimental.pallas import tpu_sc as plsc

N = 10000
E = 320000
NFEAT = 128
NHID = 50
NOUT = 50
K = 16
TEMP = 50.0

FP = 64
NW = 32
E2 = E + 512
EPW = E2 // NW
EDUMMY = E

_SIZES = ([10240] * 4 + [9920] * 12) * 2
_BOUNDS = np.cumsum(_SIZES)[:-1]
NSEG = N + 31
J2 = 10240
JDUMMY = NSEG
J3 = 10240
KS = 192
K2S = 40
JPW = J2 // NW
CH = 80


def _mm_body(x_ref, w_ref, o_ref):
    o_ref[...] = jnp.dot(x_ref[...], w_ref[...],
                         preferred_element_type=jnp.float32)


def _tc_matmul(x, w):
    m = x.shape[0]
    return pl.pallas_call(
        _mm_body,
        out_shape=jax.ShapeDtypeStruct((m, w.shape[1]), jnp.float32),
    )(x, w)


def _mid_body(a_ref, b_ref, w_ref, o_ref):
    h = jnp.maximum(a_ref[...] + b_ref[...], 0.0)
    o_ref[...] = jnp.dot(h, w_ref[...],
                         preferred_element_type=jnp.float32)


def _tc_mid(agg, b, w):
    return pl.pallas_call(
        _mid_body,
        out_shape=jax.ShapeDtypeStruct((N, FP), jnp.float32),
    )(agg, b, w)


def _softmax_bfly(logits):
    m = jnp.max(logits, axis=1, keepdims=True)
    e = jnp.exp(logits - m)
    s = e[:, :8] + e[:, 8:]
    s = s[:, :4] + s[:, 4:8]
    s = s[:, :2] + s[:, 2:4]
    s = s[:, :1] + s[:, 1:2]
    return e / s


def _iter_body(d_ref, m_ref, r_ref, cm_ref):
    data = d_ref[...]
    dist = lax.dot_general(data, m_ref[...], (((1,), (1,)), ((), ())))
    r = _softmax_bfly(TEMP * dist)
    r_ref[...] = r
    cm_ref[...] = lax.dot_general(r, data, (((0,), (0,)), ((), ())))


def _pal_iter(data, mu):
    return pl.pallas_call(_iter_body, out_shape=(
        jax.ShapeDtypeStruct((N, K), jnp.float32),
        jax.ShapeDtypeStruct((K, NOUT), jnp.float32)))(data, mu)


def _final_body(d_ref, m_ref, r_ref, dist_ref):
    data = d_ref[...]
    dist = lax.dot_general(data, m_ref[...], (((1,), (1,)), ((), ())))
    dist_ref[...] = dist
    r_ref[...] = _softmax_bfly(TEMP * dist)


def _pal_final(data, mu):
    return pl.pallas_call(_final_body, out_shape=(
        jax.ShapeDtypeStruct((N, K), jnp.float32),
        jax.ShapeDtypeStruct((N, K), jnp.float32)))(data, mu)


_MESH = plsc.VectorSubcoreMesh(core_axis_name="c", subcore_axis_name="s")
_SC_PARAMS = pltpu.CompilerParams(use_tc_tiling_on_sc=False)


def _sc_gather_scale(support, src_s, ew_s):
    @functools.partial(
        pl.kernel,
        mesh=_MESH,
        compiler_params=_SC_PARAMS,
        out_type=jax.ShapeDtypeStruct((E2, FP), jnp.float32),
        scratch_types=[
            pltpu.VMEM((EPW,), jnp.int32),
            pltpu.VMEM((EPW,), jnp.float32),
            pltpu.VMEM((CH, FP), jnp.float32),
        ],
    )
    def k(sup_hbm, src_hbm, ew_hbm, out_hbm, src_v, ew_v, buf_v):
        cid = lax.axis_index("c")
        sid = lax.axis_index("s")
        wid = cid * 16 + sid
        base = wid * EPW
        pltpu.sync_copy(src_hbm.at[pl.ds(base, EPW)], src_v)
        pltpu.sync_copy(ew_hbm.at[pl.ds(base, EPW)], ew_v)

        def do_chunk(off, n):
            pltpu.sync_copy(sup_hbm.at[src_v.at[pl.ds(off, n)]],
                            buf_v.at[pl.ds(0, n)])

            @pl.loop(0, n // 16)
            def _(g):
                wv = ew_v[pl.ds(off + g * 16, 16)]
                for l in range(16):
                    w = wv[l]
                    e = g * 16 + l
                    for q in range(FP // 16):
                        sl = (e, pl.ds(q * 16, 16))
                        buf_v[sl] = buf_v[sl] * w

            pltpu.sync_copy(buf_v.at[pl.ds(0, n)],
                            out_hbm.at[pl.ds(base + off, n)])

        @pl.loop(0, 125)
        def _(c):
            do_chunk(c * CH, CH)

        do_chunk(125 * CH, 16)

    return k(support, src_s, ew_s)


def _sc_fold(table, m3d, kmax16, zeros, jout, ks):
    @functools.partial(
        pl.kernel,
        mesh=_MESH,
        compiler_params=_SC_PARAMS,
        out_type=jax.ShapeDtypeStruct((jout, FP), jnp.float32),
        scratch_types=[
            pltpu.VMEM((ks, JPW), jnp.int32),
            pltpu.VMEM((JPW, FP), jnp.float32),
            pltpu.VMEM((CH, FP), jnp.float32),
            pltpu.VMEM((16,), jnp.int32),
        ],
    )
    def k(tab_hbm, m_hbm, kmax_hbm, z_hbm, out_hbm, m_v, acc_v, buf_v, kv_v):
        cid = lax.axis_index("c")
        sid = lax.axis_index("s")
        wid = cid * 16 + sid
        pltpu.sync_copy(m_hbm.at[wid], m_v)
        pltpu.sync_copy(kmax_hbm, kv_v)
        pltpu.sync_copy(z_hbm, acc_v)
        khi = kv_v[pl.ds(0, 16)][0]

        @pl.loop(0, khi)
        def _(kk):
            for c in range(JPW // CH):
                pltpu.sync_copy(
                    tab_hbm.at[m_v.at[kk, pl.ds(c * CH, CH)]], buf_v)

                @pl.loop(0, CH)
                def _(e):
                    for q in range(FP // 16):
                        sl = (c * CH + e, pl.ds(q * 16, 16))
                        sb = (e, pl.ds(q * 16, 16))
                        acc_v[sl] = acc_v[sl] + buf_v[sb]

        pltpu.sync_copy(acc_v, out_hbm.at[pl.ds(wid * JPW, JPW)])

    return k(table, m3d, kmax16, zeros)


def _segsum(support, src_s, ew_s, m2, kmax2_16, m3, kmax3_16, zeros):
    msg = _sc_gather_scale(support, src_s, ew_s)
    parts = _sc_fold(msg, m2, kmax2_16, zeros, J2, KS)
    agg = _sc_fold(parts, m3, kmax3_16, zeros, J3, K2S)
    return agg[:N]


def kernel(x, edge_index, edge_weight, W1, b1, W2, b2, init, num_iter):
    src = edge_index[0]
    dst = edge_index[1]

    perm = jnp.argsort(dst, stable=True)
    dst_s = dst[perm]
    src_s = jnp.concatenate(
        [src[perm], jnp.zeros((E2 - E,), jnp.int32)])
    ew_s = jnp.concatenate(
        [edge_weight[perm], jnp.zeros((E2 - E,), jnp.float32)])

    rowptr = jnp.searchsorted(dst_s, jnp.arange(N + 1, dtype=jnp.int32)
                              ).astype(jnp.int32)
    bounds = jnp.asarray(_BOUNDS, jnp.int32)
    splits = jnp.sort(jnp.concatenate([rowptr[:N], bounds]))
    ends = jnp.concatenate([splits[1:], jnp.array([E], jnp.int32)])
    seglen = ends - splits

    k_ar = jnp.arange(KS, dtype=jnp.int32)[:, None]
    m2 = jnp.where(k_ar < seglen[None, :], splits[None, :] + k_ar, EDUMMY)
    m2 = jnp.pad(m2, ((0, 0), (0, J2 - NSEG)), constant_values=EDUMMY)
    m2 = m2.reshape(KS, NW, JPW).transpose(1, 0, 2)
    kmax2 = jnp.minimum(jnp.max(seglen), KS)
    kmax2_16 = jnp.full((16,), kmax2, jnp.int32)

    firsts = jnp.searchsorted(splits, rowptr, side="left").astype(jnp.int32)
    nseg = firsts[1:] - firsts[:-1]
    k2_ar = jnp.arange(K2S, dtype=jnp.int32)[:, None]
    m3 = jnp.where(k2_ar < nseg[None, :], firsts[None, :-1] + k2_ar, JDUMMY)
    m3 = jnp.pad(m3, ((0, 0), (0, J3 - N)), constant_values=JDUMMY)
    m3 = m3.reshape(K2S, NW, JPW).transpose(1, 0, 2)
    kmax3 = jnp.minimum(jnp.max(nseg), K2S)
    kmax3_16 = jnp.full((16,), kmax3, jnp.int32)

    zeros = jnp.zeros((JPW, FP), jnp.float32)

    W1p = jnp.pad(W1, ((0, 0), (0, FP - NHID)))
    W2p = jnp.pad(W2, ((0, FP - NHID), (0, FP - NOUT)))
    b1p = jnp.pad(b1, (0, FP - NHID)).reshape(1, FP)

    support1 = _tc_matmul(x, W1p)
    agg1 = _segsum(support1, src_s, ew_s, m2, kmax2_16, m3, kmax3_16, zeros)
    support2 = _tc_mid(agg1, b1p, W2p)
    agg2 = _segsum(support2, src_s, ew_s, m2, kmax2_16, m3, kmax3_16, zeros)
    emb = agg2[:, :NOUT] + b2[None, :]

    data = emb / jnp.linalg.norm(emb, axis=1, keepdims=True)

    def body(_, mu):
        r, cm = _pal_iter(data, mu)
        cr = r.sum(axis=0)
        return cm / cr[:, None]

    mu_i = lax.fori_loop(0, num_iter, body, init)
    mu = body(0, lax.stop_gradient(mu_i))
    r, dist = _pal_final(data, mu)
    return (mu, r, emb, dist)
